# baseline (device time: 31763 ns/iter reference)
import jax
import jax.numpy as jnp
from jax import lax
from jax.experimental import pallas as pl
from jax.experimental.pallas import tpu as pltpu

CH = 64


def kernel(x, dest):
    t, d = x.shape
    maxc = t // CH

    d0 = dest == 0
    cum0 = jnp.cumsum(d0.astype(jnp.int32))
    c0 = cum0[-1]
    iota = jnp.arange(t, dtype=jnp.int32)
    rank = jnp.where(d0, cum0 - 1, c0 + iota - cum0)
    inv = jnp.zeros((t,), jnp.int32).at[rank].set(iota, unique_indices=True)
    both = jnp.concatenate([inv, jnp.roll(inv, t - c0)])
    big = x[both]
    xs, send_buf = big[:t], big[t:]
    cnt = jnp.reshape(c0, (1,))

    def body(cnt_ref, xs_ref, send_ref, out_ref, send_sems, recv_sems):
        mx = lax.axis_index("x")
        my = lax.axis_index("y")
        mz = lax.axis_index("z")
        partner = (1 - mx, my, mz)

        barrier_sem = pltpu.get_barrier_semaphore()
        pl.semaphore_signal(
            barrier_sem,
            inc=1,
            device_id=partner,
            device_id_type=pl.DeviceIdType.MESH,
        )
        pl.semaphore_wait(barrier_sem, 1)

        c0v = cnt_ref[0]
        is0 = mx == 0
        cs = jnp.where(is0, t - c0v, c0v)
        n = (cs + CH - 1) // CH

        def chunk_copy(k):
            off = jnp.where(is0, k * CH, t - (k + 1) * CH)
            return pltpu.make_async_remote_copy(
                src_ref=send_ref.at[pl.ds(off, CH), :],
                dst_ref=out_ref.at[pl.ds(off, CH), :],
                send_sem=send_sems.at[k],
                recv_sem=recv_sems.at[k],
                device_id=partner,
                device_id_type=pl.DeviceIdType.MESH,
            )

        for k in range(maxc):
            @pl.when(k < n)
            def _(k=k):
                chunk_copy(k).start()

        for k in range(maxc):
            @pl.when(k < n)
            def _(k=k):
                chunk_copy(k).wait_recv()

        rows = lax.broadcasted_iota(jnp.int32, (t, 1), 0)
        s = 1 - 2 * mx
        own_mask = (s * rows) < (s * c0v + mx)
        out_ref[:, :] = jnp.where(own_mask, xs_ref[:, :], out_ref[:, :])

        for k in range(maxc):
            @pl.when(k < n)
            def _(k=k):
                chunk_copy(k).wait_send()

    return pl.pallas_call(
        body,
        out_shape=jax.ShapeDtypeStruct((t, d), x.dtype),
        in_specs=[
            pl.BlockSpec(memory_space=pltpu.SMEM),
            pl.BlockSpec(memory_space=pltpu.VMEM),
            pl.BlockSpec(memory_space=pltpu.VMEM),
        ],
        out_specs=pl.BlockSpec(memory_space=pltpu.VMEM),
        scratch_shapes=[
            pltpu.SemaphoreType.DMA((maxc,)),
            pltpu.SemaphoreType.DMA((maxc,)),
        ],
        compiler_params=pltpu.CompilerParams(collective_id=0),
    )(cnt, xs, send_buf)


# device time: 30138 ns/iter; 1.0539x vs baseline; 1.0539x over previous
import jax
import jax.numpy as jnp
from jax import lax
from jax.experimental import pallas as pl
from jax.experimental.pallas import tpu as pltpu

CH = 64


def kernel(x, dest):
    t, d = x.shape
    maxc = t // CH

    iota = jnp.arange(t, dtype=jnp.int32)
    packed = dest.astype(jnp.int32) * (2 * t) + iota
    inv = jnp.sort(packed) & (2 * t - 1)
    c0 = jnp.sum(dest == 0).astype(jnp.int32)
    both = jnp.concatenate([inv, jnp.roll(inv, t - c0)])
    big = x[both]
    cnt = jnp.reshape(c0, (1,))

    def body(cnt_ref, big_ref, out_ref, pad_ref, send_sems, recv_sems,
             pad_ssem, pad_rsem):
        mx = lax.axis_index("x")
        my = lax.axis_index("y")
        mz = lax.axis_index("z")
        partner = (1 - mx, my, mz)

        barrier_sem = pltpu.get_barrier_semaphore()
        pl.semaphore_signal(
            barrier_sem,
            inc=1,
            device_id=partner,
            device_id_type=pl.DeviceIdType.MESH,
        )
        pl.semaphore_wait(barrier_sem, 1)

        c0v = cnt_ref[0]
        is0 = mx == 0
        cs = jnp.where(is0, t - c0v, c0v)
        n = (cs + CH - 1) // CH

        def off_at(k):
            return jnp.where(is0, k * CH, t - (k + 1) * CH)

        def full_chunk(k):
            off = off_at(k)
            return pltpu.make_async_remote_copy(
                src_ref=big_ref.at[pl.ds(t + off, CH), :],
                dst_ref=out_ref.at[pl.ds(off, CH), :],
                send_sem=send_sems.at[k],
                recv_sem=recv_sems.at[k],
                device_id=partner,
                device_id_type=pl.DeviceIdType.MESH,
            )

        def pad_chunk():
            off = off_at(n - 1)
            return pltpu.make_async_remote_copy(
                src_ref=big_ref.at[pl.ds(t + off, CH), :],
                dst_ref=pad_ref,
                send_sem=pad_ssem,
                recv_sem=pad_rsem,
                device_id=partner,
                device_id_type=pl.DeviceIdType.MESH,
            )

        for k in range(maxc):
            @pl.when(k < n - 1)
            def _(k=k):
                full_chunk(k).start()

        @pl.when(n > 0)
        def _():
            pad_chunk().start()

        for k in range(maxc):
            @pl.when(k < maxc - n)
            def _(k=k):
                off = off_at(k)
                out_ref[pl.ds(off, CH), :] = big_ref[pl.ds(off, CH), :]

        for k in range(maxc):
            @pl.when(k < n - 1)
            def _(k=k):
                full_chunk(k).wait_recv()

        @pl.when(n > 0)
        def _():
            pad_chunk().wait_recv()
            qb = jnp.where(is0, t - n * CH, (n - 1) * CH)
            rows = qb + lax.broadcasted_iota(jnp.int32, (CH, 1), 0)
            s = 1 - 2 * mx
            own_mask = (s * rows) < (s * c0v + mx)
            out_ref[pl.ds(qb, CH), :] = jnp.where(
                own_mask, big_ref[pl.ds(qb, CH), :], pad_ref[:, :]
            )

        for k in range(maxc):
            @pl.when(k < n - 1)
            def _(k=k):
                full_chunk(k).wait_send()

        @pl.when(n > 0)
        def _():
            pad_chunk().wait_send()

    return pl.pallas_call(
        body,
        out_shape=jax.ShapeDtypeStruct((t, d), x.dtype),
        in_specs=[
            pl.BlockSpec(memory_space=pltpu.SMEM),
            pl.BlockSpec(memory_space=pltpu.VMEM),
        ],
        out_specs=pl.BlockSpec(memory_space=pltpu.VMEM),
        scratch_shapes=[
            pltpu.VMEM((CH, d), x.dtype),
            pltpu.SemaphoreType.DMA((maxc,)),
            pltpu.SemaphoreType.DMA((maxc,)),
            pltpu.SemaphoreType.DMA,
            pltpu.SemaphoreType.DMA,
        ],
        compiler_params=pltpu.CompilerParams(collective_id=0),
    )(cnt, big)
